# Initial kernel scaffold; baseline (speedup 1.0000x reference)
import jax
import jax.numpy as jnp
from jax.experimental import pallas as pl


def kernel(x, prototype_bank):
    raise NotImplementedError("write your pallas kernel here")


# speedup vs baseline: 1.8433x; 1.8433x over previous
import jax
import jax.numpy as jnp
from jax.experimental import pallas as pl
from jax.experimental.pallas import tpu as pltpu

B, C, H, W = 16, 256, 32, 32
HW = H * W
K = 1024


def _match_kernel(x_ref, bank_ref, bank_t_ref, out_ref, idx_ref, pn_ref):
    b = pl.program_id(0)

    @pl.when(b == 0)
    def _():
        bank = bank_ref[...]
        norm = jnp.sqrt(jnp.sum(bank * bank, axis=1, keepdims=True))
        pn_ref[...] = bank / jnp.maximum(norm, 1e-12)

    xb = x_ref[0]
    xnorm = jnp.sqrt(jnp.sum(xb * xb, axis=0, keepdims=True))
    xn = xb / jnp.maximum(xnorm, 1e-12)

    sim = jnp.dot(pn_ref[...], xn, preferred_element_type=jnp.float32)

    m = jnp.max(sim, axis=0, keepdims=True)
    iota_k = jax.lax.broadcasted_iota(jnp.int32, (K, HW), 0)
    masked = jnp.where(sim == m, iota_k, K)
    idx = jnp.min(masked, axis=0, keepdims=True)
    idx_ref[0] = idx

    onehot = (iota_k == idx).astype(jnp.float32)
    out_ref[0] = jnp.dot(bank_t_ref[...], onehot,
                         preferred_element_type=jnp.float32)


def kernel(x, prototype_bank):
    xf = x.reshape(B, C, HW)
    bank_t = prototype_bank.T

    out, idx3 = pl.pallas_call(
        _match_kernel,
        grid=(B,),
        in_specs=[
            pl.BlockSpec((1, C, HW), lambda b: (b, 0, 0)),
            pl.BlockSpec((K, C), lambda b: (0, 0)),
            pl.BlockSpec((C, K), lambda b: (0, 0)),
        ],
        out_specs=[
            pl.BlockSpec((1, C, HW), lambda b: (b, 0, 0)),
            pl.BlockSpec((1, 1, HW), lambda b: (b, 0, 0)),
        ],
        out_shape=[
            jax.ShapeDtypeStruct((B, C, HW), jnp.float32),
            jax.ShapeDtypeStruct((B, 1, HW), jnp.int32),
        ],
        scratch_shapes=[pltpu.VMEM((K, C), jnp.float32)],
    )(xf, prototype_bank, bank_t)

    return out.reshape(B, C, H, W), idx3.reshape(B, HW)
